# baseline (speedup 1.0000x reference)
import functools

import jax
import jax.numpy as jnp
from jax import lax
from jax.experimental import pallas as pl
from jax.experimental.pallas import tpu as pltpu
from jax.experimental.pallas import tpu_sc as plsc

_N = 8 * 512 * 512
_NW = 32
_SPAN = _N // _NW
_CHUNK = 8192
_NCHUNK = _SPAN // _CHUNK
_GROUPS = _CHUNK // 16
_UNROLL = 8

_mesh = plsc.VectorSubcoreMesh(core_axis_name="c", subcore_axis_name="s",
                               num_cores=2)


def _wid():
    return lax.axis_index("s") * 2 + lax.axis_index("c")


@functools.partial(
    pl.kernel, mesh=_mesh,
    out_type=jax.ShapeDtypeStruct((_NW * 80,), jnp.float32),
    scratch_types=[
        pltpu.VMEM((_CHUNK,), jnp.float32),
        pltpu.VMEM((_CHUNK,), jnp.float32),
        pltpu.VMEM((80,), jnp.float32),
    ],
)
def _sc_pass1(x_hbm, y_hbm, out_hbm, xv, yv, accv):
    base = _wid() * _SPAN
    z = jnp.zeros((16,), jnp.float32)

    def chunk_body(ci, accs):
        pltpu.sync_copy(x_hbm.at[pl.ds(base + ci * _CHUNK, _CHUNK)], xv)
        pltpu.sync_copy(y_hbm.at[pl.ds(base + ci * _CHUNK, _CHUNK)], yv)

        def grp_body(g, accs):
            s0, s1, s2, s3, s4 = accs
            for u in range(_UNROLL):
                off = (g * _UNROLL + u) * 16
                xs = xv[pl.ds(off, 16)]
                ys = yv[pl.ds(off, 16)]
                disp = 1.0 / jnp.maximum(xs, 1e-6)
                mask = (xs > 0.1) & (xs < 100.0)
                zero = jnp.zeros_like(xs)
                a = jnp.where(mask, disp, zero)
                b = jnp.where(mask, ys, zero)
                mf = jnp.where(mask, 1.0, zero)
                s0 = s0 + mf
                s1 = s1 + a
                s2 = s2 + b
                s3 = s3 + a * b
                s4 = s4 + b * b
            return (s0, s1, s2, s3, s4)

        return lax.fori_loop(0, _GROUPS // _UNROLL, grp_body, accs)

    accs = lax.fori_loop(0, _NCHUNK, chunk_body, (z, z, z, z, z))
    for j in range(5):
        accv[pl.ds(j * 16, 16)] = accs[j]
    pltpu.sync_copy(accv, out_hbm.at[pl.ds(_wid() * 80, 80)])


@functools.partial(
    pl.kernel, mesh=_mesh,
    out_type=jax.ShapeDtypeStruct((_NW * 16,), jnp.float32),
    scratch_types=[
        pltpu.VMEM((_CHUNK,), jnp.float32),
        pltpu.VMEM((_CHUNK,), jnp.float32),
        pltpu.VMEM((_NW * 80,), jnp.float32),
        pltpu.VMEM((16,), jnp.float32),
    ],
)
def _sc_pass2(x_hbm, y_hbm, p_hbm, out_hbm, xv, yv, pv, lv):
    base = _wid() * _SPAN
    pltpu.sync_copy(p_hbm, pv)
    z = jnp.zeros((16,), jnp.float32)

    def comb_body(w, accs):
        s0, s1, s2, s3, s4 = accs
        s0 = s0 + pv[pl.ds(w * 80, 16)]
        s1 = s1 + pv[pl.ds(w * 80 + 16, 16)]
        s2 = s2 + pv[pl.ds(w * 80 + 32, 16)]
        s3 = s3 + pv[pl.ds(w * 80 + 48, 16)]
        s4 = s4 + pv[pl.ds(w * 80 + 64, 16)]
        return (s0, s1, s2, s3, s4)

    sums = lax.fori_loop(0, _NW, comb_body, (z, z, z, z, z))
    totals = []
    for j in range(5):
        tot = sums[j][0]
        for k in range(1, 16):
            tot = tot + sums[j][k]
        totals.append(jnp.full((16,), tot, jnp.float32))
    cnt = jnp.maximum(totals[0], 1.0)
    mean_r = totals[1] / cnt
    mean_p = totals[2] / cnt
    mean_rp = totals[3] / cnt
    mean_pp = totals[4] / cnt
    covar = mean_rp - mean_r * mean_p
    var_p = mean_pp - mean_p * mean_p
    s = jnp.maximum(covar / (var_p + 1e-8), 1e-4)
    t = mean_r - s * mean_p

    def chunk_body(ci, lacc):
        pltpu.sync_copy(x_hbm.at[pl.ds(base + ci * _CHUNK, _CHUNK)], xv)
        pltpu.sync_copy(y_hbm.at[pl.ds(base + ci * _CHUNK, _CHUNK)], yv)

        def grp_body(g, lacc):
            for u in range(_UNROLL):
                off = (g * _UNROLL + u) * 16
                xs = xv[pl.ds(off, 16)]
                ys = yv[pl.ds(off, 16)]
                disp = 1.0 / jnp.maximum(xs, 1e-6)
                mask = (xs > 0.1) & (xs < 100.0)
                zero = jnp.zeros_like(xs)
                a = jnp.where(mask, disp, zero)
                b = jnp.where(mask, ys, zero)
                tm = jnp.where(mask, t, 0.0)
                lacc = lacc + jnp.abs(a - s * b - tm)
            return lacc

        return lax.fori_loop(0, _GROUPS // _UNROLL, grp_body, lacc)

    lacc = lax.fori_loop(0, _NCHUNK, chunk_body, z)
    lv[...] = lacc / cnt
    pltpu.sync_copy(lv, out_hbm.at[pl.ds(_wid() * 16, 16)])


def kernel(render_depth, prior_disp):
    x = render_depth.reshape(_N)
    y = prior_disp.reshape(_N)
    partials = _sc_pass1(x, y)
    per_worker = _sc_pass2(x, y, partials)
    return jnp.sum(per_worker)


# speedup vs baseline: 5.2154x; 5.2154x over previous
import jax
import jax.numpy as jnp
from jax.experimental import pallas as pl
from jax.experimental.pallas import tpu as pltpu

_B = 8
_H = 512
_W = 512
_CH = 8
_BH = 256
_NCH = _BH // _CH


def _loss_kernel(x_ref, y_ref, o_ref, a_ref, b_ref):
    i = pl.program_id(0)
    base = i * _BH
    for k in range(_NCH):
        r = k * _CH
        xs = x_ref[0, 0, r:r + _CH, :]
        ys = y_ref[0, 0, r:r + _CH, :]
        disp = 1.0 / jnp.maximum(xs, 1e-6)
        mask = (xs > 0.1) & (xs < 100.0)
        zero = jnp.zeros_like(xs)
        a_ref[pl.ds(base + r, _CH), :] = jnp.where(mask, disp, zero)
        b_ref[pl.ds(base + r, _CH), :] = jnp.where(mask, ys, zero)

    @pl.when(i == 2 * _B - 1)
    def _finish():
        def sbody(k, carry):
            s_m, s_a, s_b, s_ab, s_bb = carry
            r0 = k * 16
            af = a_ref[pl.ds(r0, 16), :]
            bf = b_ref[pl.ds(r0, 16), :]
            mf = jnp.where(af > 0.0, 1.0, 0.0)
            s_m = s_m + mf
            s_a = s_a + af
            s_b = s_b + bf
            s_ab = s_ab + af * bf
            s_bb = s_bb + bf * bf
            return (s_m, s_a, s_b, s_ab, s_bb)

        z16s = jnp.zeros((16, _W), jnp.float32)
        s_m, s_a, s_b, s_ab, s_bb = jax.lax.fori_loop(
            0, (_B * _H) // 16, sbody, (z16s,) * 5, unroll=2)
        cnt = jnp.maximum(jnp.sum(s_m), 1.0)
        mean_r = jnp.sum(s_a) / cnt
        mean_p = jnp.sum(s_b) / cnt
        mean_rp = jnp.sum(s_ab) / cnt
        mean_pp = jnp.sum(s_bb) / cnt
        covar = mean_rp - mean_r * mean_p
        var_p = mean_pp - mean_p * mean_p
        s = jnp.maximum(covar / (var_p + 1e-8), 1e-4)
        t = mean_r - s * mean_p

        def term(af, bf):
            tm = jnp.where(af > 0.0, t, 0.0)
            return jnp.abs(af - s * bf - tm)

        def body(k, carry):
            l0, l1 = carry
            r0 = k * 32
            l0 = l0 + term(a_ref[pl.ds(r0, 16), :], b_ref[pl.ds(r0, 16), :])
            l1 = l1 + term(a_ref[pl.ds(r0 + 16, 16), :],
                           b_ref[pl.ds(r0 + 16, 16), :])
            return (l0, l1)

        z16 = jnp.zeros((16, _W), jnp.float32)
        l0, l1 = jax.lax.fori_loop(
            0, (_B * _H) // 32, body, (z16, z16), unroll=2)
        o_ref[...] = jnp.full((1, 1), jnp.sum(l0 + l1) / cnt, jnp.float32)


def kernel(render_depth, prior_disp):
    out = pl.pallas_call(
        _loss_kernel,
        grid=(2 * _B,),
        in_specs=[
            pl.BlockSpec((1, 1, _BH, _W), lambda i: (i // 2, 0, i % 2, 0)),
            pl.BlockSpec((1, 1, _BH, _W), lambda i: (i // 2, 0, i % 2, 0)),
        ],
        out_specs=pl.BlockSpec((1, 1), lambda i: (0, 0)),
        out_shape=jax.ShapeDtypeStruct((1, 1), jnp.float32),
        scratch_shapes=[
            pltpu.VMEM((_B * _H, _W), jnp.float32),
            pltpu.VMEM((_B * _H, _W), jnp.float32),
        ],
    )(render_depth, prior_disp)
    return out.reshape(())


# speedup vs baseline: 5.9547x; 1.1418x over previous
import jax
import jax.numpy as jnp
from jax.experimental import pallas as pl
from jax.experimental.pallas import tpu as pltpu

_B = 8
_H = 512
_W = 512
_CH = 8
_NCH = _H // _CH


def _fold(v):
    acc = v[:, 0:128]
    for j in range(1, 4):
        acc = acc + v[:, j * 128:(j + 1) * 128]
    return acc


def _loss_kernel(x_ref, y_ref, o_ref, a_ref, b_ref, acc_ref):
    i = pl.program_id(0)

    @pl.when(i == 0)
    def _init():
        acc_ref[...] = jnp.zeros((40, 128), jnp.float32)

    base = i * _H
    s_m = jnp.zeros((_CH, 128), jnp.float32)
    s_a = jnp.zeros((_CH, 128), jnp.float32)
    s_b = jnp.zeros((_CH, 128), jnp.float32)
    s_ab = jnp.zeros((_CH, 128), jnp.float32)
    s_bb = jnp.zeros((_CH, 128), jnp.float32)
    for k in range(_NCH):
        r = k * _CH
        xs = x_ref[0, 0, r:r + _CH, :]
        ys = y_ref[0, 0, r:r + _CH, :]
        disp = 1.0 / jnp.maximum(xs, 1e-6)
        mask = (xs > 0.1) & (xs < 100.0)
        zero = jnp.zeros_like(xs)
        a = jnp.where(mask, disp, zero)
        b = jnp.where(mask, ys, zero)
        mf = jnp.where(mask, 1.0, zero)
        a_ref[pl.ds(base + r, _CH), :] = a
        b_ref[pl.ds(base + r, _CH), :] = b
        s_m = s_m + _fold(mf)
        s_a = s_a + _fold(a)
        s_b = s_b + _fold(b)
        s_ab = s_ab + _fold(a * b)
        s_bb = s_bb + _fold(b * b)
    acc_ref[0:8, :] += s_m
    acc_ref[8:16, :] += s_a
    acc_ref[16:24, :] += s_b
    acc_ref[24:32, :] += s_ab
    acc_ref[32:40, :] += s_bb

    @pl.when(i == _B - 1)
    def _finish():
        cnt = jnp.maximum(jnp.sum(acc_ref[0:8, :]), 1.0)
        mean_r = jnp.sum(acc_ref[8:16, :]) / cnt
        mean_p = jnp.sum(acc_ref[16:24, :]) / cnt
        mean_rp = jnp.sum(acc_ref[24:32, :]) / cnt
        mean_pp = jnp.sum(acc_ref[32:40, :]) / cnt
        covar = mean_rp - mean_r * mean_p
        var_p = mean_pp - mean_p * mean_p
        s = jnp.maximum(covar / (var_p + 1e-8), 1e-4)
        t = mean_r - s * mean_p

        def body(k, l_acc):
            af = a_ref[pl.ds(k * _CH, _CH), :]
            bf = b_ref[pl.ds(k * _CH, _CH), :]
            tm = jnp.where(af > 0.0, t, 0.0)
            return l_acc + _fold(jnp.abs(af - s * bf - tm))

        l_acc = jax.lax.fori_loop(
            0, (_B * _H) // _CH, body, jnp.zeros((_CH, 128), jnp.float32))
        o_ref[...] = jnp.full((1, 1), jnp.sum(l_acc) / cnt, jnp.float32)


def kernel(render_depth, prior_disp):
    out = pl.pallas_call(
        _loss_kernel,
        grid=(_B,),
        in_specs=[
            pl.BlockSpec((1, 1, _H, _W), lambda i: (i, 0, 0, 0)),
            pl.BlockSpec((1, 1, _H, _W), lambda i: (i, 0, 0, 0)),
        ],
        out_specs=pl.BlockSpec((1, 1), lambda i: (0, 0)),
        out_shape=jax.ShapeDtypeStruct((1, 1), jnp.float32),
        scratch_shapes=[
            pltpu.VMEM((_B * _H, _W), jnp.float32),
            pltpu.VMEM((_B * _H, _W), jnp.float32),
            pltpu.VMEM((40, 128), jnp.float32),
        ],
    )(render_depth, prior_disp)
    return out.reshape(())
